# baseline (speedup 1.0000x reference)
import functools

import jax
import jax.numpy as jnp
from jax import lax
from jax.experimental import pallas as pl
from jax.experimental.pallas import tpu as pltpu
from jax.experimental.pallas import tpu_sc as plsc

_NC = 2
_NS = 16
_NW = _NC * _NS
_LANES = 128

_NBUF = 4


def _transpose_sc(table_t, *, vocab, d_model):
    n_cols = (vocab + _LANES - 1) // _LANES
    vocab_pad = n_cols * _LANES
    base_cols, extra = divmod(n_cols, _NW)
    mesh = plsc.VectorSubcoreMesh(core_axis_name="c", subcore_axis_name="s")

    @functools.partial(
        pl.kernel,
        out_type=jax.ShapeDtypeStruct((vocab_pad, _LANES), jnp.float32),
        mesh=mesh,
        scratch_types=[
            pltpu.VMEM((2, d_model, _LANES), jnp.float32),
            pltpu.VMEM((2, _LANES, _LANES), jnp.float32),
            pltpu.SemaphoreType.DMA((2,)),
            pltpu.SemaphoreType.DMA((2,)),
        ],
        compiler_params=pltpu.CompilerParams(
            use_tc_tiling_on_sc=True,
            disable_bounds_checks=True,
            needs_layout_passes=False,
        ),
    )
    def k(wt_hbm, out_hbm, vbufs, tbufs, isems, osems):
        wid = lax.axis_index("s") * _NC + lax.axis_index("c")
        my_cols = base_cols + jnp.where(wid < extra, 1, 0)
        col0 = wid * base_cols + jnp.minimum(wid, extra)
        iota = lax.iota(jnp.int32, 16)

        def fetch(slot, t):
            pltpu.async_copy(
                wt_hbm.at[pl.ds(0, d_model), pl.ds(t * _LANES, _LANES)],
                vbufs.at[slot],
                isems.at[slot],
            )

        def fetch_wait(slot):
            pltpu.make_async_copy(
                wt_hbm.at[pl.ds(0, d_model), pl.ds(0, _LANES)],
                vbufs.at[slot],
                isems.at[slot],
            ).wait()

        def store_wait(slot):
            pltpu.make_async_copy(
                tbufs.at[slot], out_hbm.at[pl.ds(0, _LANES)], osems.at[slot]
            ).wait()

        @pl.when(my_cols > 0)
        def _():
            fetch(0, col0)

        n_outer = (base_cols + 2) // 2

        def body(t, carry):
            for slot in range(2):
                i = 2 * t + slot

                @pl.when(i < my_cols)
                def _():
                    @pl.when(i + 1 < my_cols)
                    def _():
                        fetch(1 - slot, col0 + i + 1)

                    fetch_wait(slot)

                    @pl.when(i >= 2)
                    def _():
                        store_wait(slot)

                    for c in range(d_model):
                        cvec = jnp.full((16,), c, jnp.int32)
                        for l in range(_LANES // 16):
                            x = vbufs[slot, c, pl.ds(16 * l, 16)]
                            plsc.store_scatter(
                                tbufs.at[slot], [16 * l + iota, cvec], x
                            )
                    pltpu.async_copy(
                        tbufs.at[slot],
                        out_hbm.at[pl.ds((col0 + i) * _LANES, _LANES)],
                        osems.at[slot],
                    )

            return carry

        lax.fori_loop(0, n_outer, body, 0)

        for back in range(2, 0, -1):
            @pl.when(my_cols >= back)
            def _():
                slot_t = lax.rem(my_cols - back, 2)
                pltpu.make_async_copy(
                    tbufs.at[slot_t], out_hbm.at[pl.ds(0, _LANES)], osems.at[slot_t]
                ).wait()

    return k(table_t)


def _gather_sc(table, idx3, *, rows_per_w, seq, d_model):
    half = seq // 2
    n_groups = 2 * rows_per_w
    n_rows = _NW * rows_per_w
    mesh = plsc.VectorSubcoreMesh(core_axis_name="c", subcore_axis_name="s")
    n_main = n_groups // _NBUF - 1

    @functools.partial(
        pl.kernel,
        out_type=jax.ShapeDtypeStruct((n_rows, seq, 2 * d_model), jnp.float32),
        mesh=mesh,
        scratch_types=[
            pltpu.VMEM((n_groups, half), jnp.int32),
            pltpu.VMEM((_NBUF, half, d_model), jnp.float32),
            pltpu.SemaphoreType.DMA((_NBUF,)),
        ],
        compiler_params=pltpu.CompilerParams(use_tc_tiling_on_sc=False),
    )
    def k(table_hbm, idx_hbm, out_hbm, idx_v, bufs, gsems):
        wid = lax.axis_index("s") * _NC + lax.axis_index("c")
        row0 = wid * rows_per_w
        pltpu.sync_copy(idx_hbm.at[wid], idx_v)

        for b in range(_NBUF):
            pltpu.async_copy(table_hbm.at[idx_v.at[b]], bufs.at[b], gsems.at[b])

        def drain_store(b, g):
            pltpu.make_async_copy(
                table_hbm.at[pl.ds(0, half)], bufs.at[b], gsems.at[b]
            ).wait()
            pltpu.sync_copy(
                bufs.at[b],
                out_hbm.at[
                    row0 + g // 2, pl.ds((g % 2) * half, half), pl.ds(0, d_model)
                ],
            )

        def body(t, carry):
            for b in range(_NBUF):
                g = t * _NBUF + b
                drain_store(b, g)
                pltpu.async_copy(
                    table_hbm.at[idx_v.at[g + _NBUF]], bufs.at[b], gsems.at[b]
                )
            return carry

        lax.fori_loop(0, n_main, body, 0)

        for b in range(_NBUF):
            drain_store(b, n_main * _NBUF + b)

    return k(table, idx3)


def kernel(token_ids, weight):
    b0, s0 = token_ids.shape
    vocab, d_model = weight.shape
    rows_per_w = b0 // _NW
    scratch = _transpose_sc(weight.T, vocab=vocab, d_model=d_model)
    table2 = scratch.reshape(scratch.shape[0] * 2, d_model)
    idx3 = (token_ids * 2).reshape(_NW, 2 * rows_per_w, s0 // 2).astype(jnp.int32)
    out2 = _gather_sc(
        table2, idx3, rows_per_w=rows_per_w, seq=s0, d_model=d_model
    )
    return out2[:, :, :d_model]


# speedup vs baseline: 1.9190x; 1.9190x over previous
import functools

import jax
import jax.numpy as jnp
from jax import lax
from jax.experimental import pallas as pl
from jax.experimental.pallas import tpu as pltpu
from jax.experimental.pallas import tpu_sc as plsc

_NC = 2
_NS = 16
_NW = _NC * _NS
_LANES = 128

_NBUF = 4


_WIDEN_BLK = 2048


def _widen_tc(wt):
    d_model, vocab = wt.shape

    def body(x_ref, o_ref):
        o_ref[:, pl.ds(0, d_model)] = x_ref[...].T

    return pl.pallas_call(
        body,
        grid=(pl.cdiv(vocab, _WIDEN_BLK),),
        in_specs=[pl.BlockSpec((d_model, _WIDEN_BLK), lambda j: (0, j))],
        out_specs=pl.BlockSpec((_WIDEN_BLK, _LANES), lambda j: (j, 0)),
        out_shape=jax.ShapeDtypeStruct((vocab, _LANES), jnp.float32),
    )(wt)


def _gather_sc(table, idx3, *, rows_per_w, seq, d_model):
    half = seq // 2
    n_groups = 2 * rows_per_w
    n_rows = _NW * rows_per_w
    mesh = plsc.VectorSubcoreMesh(core_axis_name="c", subcore_axis_name="s")
    n_main = n_groups // _NBUF - 1

    @functools.partial(
        pl.kernel,
        out_type=jax.ShapeDtypeStruct((n_rows, seq, 2 * d_model), jnp.float32),
        mesh=mesh,
        scratch_types=[
            pltpu.VMEM((n_groups, half), jnp.int32),
            pltpu.VMEM((_NBUF, half, d_model), jnp.float32),
            pltpu.SemaphoreType.DMA((_NBUF,)),
        ],
        compiler_params=pltpu.CompilerParams(use_tc_tiling_on_sc=False),
    )
    def k(table_hbm, idx_hbm, out_hbm, idx_v, bufs, gsems):
        wid = lax.axis_index("s") * _NC + lax.axis_index("c")
        row0 = wid * rows_per_w
        pltpu.sync_copy(idx_hbm.at[wid], idx_v)

        for b in range(_NBUF):
            pltpu.async_copy(table_hbm.at[idx_v.at[b]], bufs.at[b], gsems.at[b])

        def drain_store(b, g):
            pltpu.make_async_copy(
                table_hbm.at[pl.ds(0, half)], bufs.at[b], gsems.at[b]
            ).wait()
            pltpu.sync_copy(
                bufs.at[b],
                out_hbm.at[
                    row0 + g // 2, pl.ds((g % 2) * half, half), pl.ds(0, d_model)
                ],
            )

        def body(t, carry):
            for b in range(_NBUF):
                g = t * _NBUF + b
                drain_store(b, g)
                pltpu.async_copy(
                    table_hbm.at[idx_v.at[g + _NBUF]], bufs.at[b], gsems.at[b]
                )
            return carry

        lax.fori_loop(0, n_main, body, 0)

        for b in range(_NBUF):
            drain_store(b, n_main * _NBUF + b)

    return k(table, idx3)


def kernel(token_ids, weight):
    b0, s0 = token_ids.shape
    vocab, d_model = weight.shape
    rows_per_w = b0 // _NW
    wide = _widen_tc(weight.T)
    table2 = wide.reshape(vocab * _LANES // d_model, d_model)
    idx3 = (token_ids * 2).reshape(_NW, 2 * rows_per_w, s0 // 2).astype(jnp.int32)
    out2 = _gather_sc(
        table2, idx3, rows_per_w=rows_per_w, seq=s0, d_model=d_model
    )
    return out2[:, :, :d_model]


# speedup vs baseline: 2.5408x; 1.3240x over previous
import functools

import jax
import jax.numpy as jnp
from jax import lax
from jax.experimental import pallas as pl
from jax.experimental.pallas import tpu as pltpu
from jax.experimental.pallas import tpu_sc as plsc

_NC = 2
_NS = 16
_NW = _NC * _NS
_LANES = 128

_NBUF = 4


_WIDEN_BLK = 8192


def _widen_tc(wt):
    d_model, vocab = wt.shape

    def body(x_ref, o_ref):
        o_ref[:, pl.ds(0, d_model)] = x_ref[...].T

    return pl.pallas_call(
        body,
        grid=(pl.cdiv(vocab, _WIDEN_BLK),),
        in_specs=[pl.BlockSpec((d_model, _WIDEN_BLK), lambda j: (0, j))],
        out_specs=pl.BlockSpec((_WIDEN_BLK, _LANES), lambda j: (j, 0)),
        out_shape=jax.ShapeDtypeStruct((vocab, _LANES), jnp.float32),
    )(wt)


def _gather_sc(table, idx3, *, rows_per_w, seq, d_model):
    half = seq // 2
    n_groups = 2 * rows_per_w
    n_rows = _NW * rows_per_w
    mesh = plsc.VectorSubcoreMesh(core_axis_name="c", subcore_axis_name="s")
    n_main = n_groups // _NBUF - 1

    @functools.partial(
        pl.kernel,
        out_type=jax.ShapeDtypeStruct((n_rows, seq, 2 * d_model), jnp.float32),
        mesh=mesh,
        scratch_types=[
            pltpu.VMEM((n_groups, half), jnp.int32),
            pltpu.VMEM((_NBUF, half, d_model), jnp.float32),
            pltpu.SemaphoreType.DMA((_NBUF,)),
        ],
        compiler_params=pltpu.CompilerParams(use_tc_tiling_on_sc=False),
    )
    def k(table_hbm, idx_hbm, out_hbm, idx_v, bufs, gsems):
        wid = lax.axis_index("s") * _NC + lax.axis_index("c")
        row0 = wid * rows_per_w
        pltpu.sync_copy(idx_hbm.at[wid], idx_v)

        for b in range(_NBUF):
            pltpu.async_copy(table_hbm.at[idx_v.at[b]], bufs.at[b], gsems.at[b])

        def drain_store(b, g):
            pltpu.make_async_copy(
                table_hbm.at[pl.ds(0, half)], bufs.at[b], gsems.at[b]
            ).wait()
            pltpu.sync_copy(
                bufs.at[b],
                out_hbm.at[
                    row0 + g // 2, pl.ds((g % 2) * half, half), pl.ds(0, d_model)
                ],
            )

        def body(t, carry):
            for b in range(_NBUF):
                g = t * _NBUF + b
                drain_store(b, g)
                pltpu.async_copy(
                    table_hbm.at[idx_v.at[g + _NBUF]], bufs.at[b], gsems.at[b]
                )
            return carry

        lax.fori_loop(0, n_main, body, 0)

        for b in range(_NBUF):
            drain_store(b, n_main * _NBUF + b)

    return k(table, idx3)


def kernel(token_ids, weight):
    b0, s0 = token_ids.shape
    vocab, d_model = weight.shape
    rows_per_w = b0 // _NW
    wide = _widen_tc(weight.T)
    table2 = wide.reshape(vocab * _LANES // d_model, d_model)
    idx3 = (token_ids * 2).reshape(_NW, 2 * rows_per_w, s0 // 2).astype(jnp.int32)
    out2 = _gather_sc(
        table2, idx3, rows_per_w=rows_per_w, seq=s0, d_model=d_model
    )
    return out2[:, :, :d_model]


# speedup vs baseline: 2.6207x; 1.0314x over previous
import functools

import jax
import jax.numpy as jnp
from jax import lax
from jax.experimental import pallas as pl
from jax.experimental.pallas import tpu as pltpu
from jax.experimental.pallas import tpu_sc as plsc

_NC = 2
_NS = 16
_NW = _NC * _NS
_LANES = 128

_NBUF = 4


_WIDEN_BLK = 16384


def _widen_tc(wt):
    d_model, vocab = wt.shape

    def body(x_ref, o_ref):
        o_ref[:, pl.ds(0, d_model)] = x_ref[...].T

    return pl.pallas_call(
        body,
        grid=(pl.cdiv(vocab, _WIDEN_BLK),),
        in_specs=[pl.BlockSpec((d_model, _WIDEN_BLK), lambda j: (0, j))],
        out_specs=pl.BlockSpec((_WIDEN_BLK, _LANES), lambda j: (j, 0)),
        out_shape=jax.ShapeDtypeStruct((vocab, _LANES), jnp.float32),
    )(wt)


def _gather_sc(table, idx3, *, rows_per_w, seq, d_model):
    half = seq // 2
    n_groups = 2 * rows_per_w
    n_rows = _NW * rows_per_w
    mesh = plsc.VectorSubcoreMesh(core_axis_name="c", subcore_axis_name="s")
    n_main = n_groups // _NBUF - 1

    @functools.partial(
        pl.kernel,
        out_type=jax.ShapeDtypeStruct((n_rows, seq, 2 * d_model), jnp.float32),
        mesh=mesh,
        scratch_types=[
            pltpu.VMEM((n_groups, half), jnp.int32),
            pltpu.VMEM((_NBUF, half, d_model), jnp.float32),
            pltpu.SemaphoreType.DMA((_NBUF,)),
        ],
        compiler_params=pltpu.CompilerParams(use_tc_tiling_on_sc=False),
    )
    def k(table_hbm, idx_hbm, out_hbm, idx_v, bufs, gsems):
        wid = lax.axis_index("s") * _NC + lax.axis_index("c")
        row0 = wid * rows_per_w
        pltpu.sync_copy(idx_hbm.at[wid], idx_v)

        for b in range(_NBUF):
            pltpu.async_copy(table_hbm.at[idx_v.at[b]], bufs.at[b], gsems.at[b])

        def drain_store(b, g):
            pltpu.make_async_copy(
                table_hbm.at[pl.ds(0, half)], bufs.at[b], gsems.at[b]
            ).wait()
            pltpu.sync_copy(
                bufs.at[b],
                out_hbm.at[
                    row0 + g // 2, pl.ds((g % 2) * half, half), pl.ds(0, d_model)
                ],
            )

        def body(t, carry):
            for b in range(_NBUF):
                g = t * _NBUF + b
                drain_store(b, g)
                pltpu.async_copy(
                    table_hbm.at[idx_v.at[g + _NBUF]], bufs.at[b], gsems.at[b]
                )
            return carry

        lax.fori_loop(0, n_main, body, 0)

        for b in range(_NBUF):
            drain_store(b, n_main * _NBUF + b)

    return k(table, idx3)


def kernel(token_ids, weight):
    b0, s0 = token_ids.shape
    vocab, d_model = weight.shape
    rows_per_w = b0 // _NW
    wide = _widen_tc(weight.T)
    table2 = wide.reshape(vocab * _LANES // d_model, d_model)
    idx3 = (token_ids * 2).reshape(_NW, 2 * rows_per_w, s0 // 2).astype(jnp.int32)
    out2 = _gather_sc(
        table2, idx3, rows_per_w=rows_per_w, seq=s0, d_model=d_model
    )
    return out2[:, :, :d_model]


# speedup vs baseline: 2.6454x; 1.0094x over previous
import functools

import jax
import jax.numpy as jnp
from jax import lax
from jax.experimental import pallas as pl
from jax.experimental.pallas import tpu as pltpu
from jax.experimental.pallas import tpu_sc as plsc

_NC = 2
_NS = 16
_NW = _NC * _NS
_LANES = 128

_NBUF = 4


_WIDEN_BLK = 32768


def _widen_tc(wt):
    d_model, vocab = wt.shape

    def body(x_ref, o_ref):
        o_ref[:, pl.ds(0, d_model)] = x_ref[...].T

    return pl.pallas_call(
        body,
        grid=(pl.cdiv(vocab, _WIDEN_BLK),),
        in_specs=[pl.BlockSpec((d_model, _WIDEN_BLK), lambda j: (0, j))],
        out_specs=pl.BlockSpec((_WIDEN_BLK, _LANES), lambda j: (j, 0)),
        out_shape=jax.ShapeDtypeStruct((vocab, _LANES), jnp.float32),
    )(wt)


def _gather_sc(table, idx3, *, rows_per_w, seq, d_model):
    half = seq // 2
    n_groups = 2 * rows_per_w
    n_rows = _NW * rows_per_w
    mesh = plsc.VectorSubcoreMesh(core_axis_name="c", subcore_axis_name="s")
    n_main = n_groups // _NBUF - 1

    @functools.partial(
        pl.kernel,
        out_type=jax.ShapeDtypeStruct((n_rows, seq, 2 * d_model), jnp.float32),
        mesh=mesh,
        scratch_types=[
            pltpu.VMEM((n_groups, half), jnp.int32),
            pltpu.VMEM((_NBUF, half, d_model), jnp.float32),
            pltpu.SemaphoreType.DMA((_NBUF,)),
        ],
        compiler_params=pltpu.CompilerParams(use_tc_tiling_on_sc=False),
    )
    def k(table_hbm, idx_hbm, out_hbm, idx_v, bufs, gsems):
        wid = lax.axis_index("s") * _NC + lax.axis_index("c")
        row0 = wid * rows_per_w
        pltpu.sync_copy(idx_hbm.at[wid], idx_v)

        for b in range(_NBUF):
            pltpu.async_copy(table_hbm.at[idx_v.at[b]], bufs.at[b], gsems.at[b])

        def drain_store(b, g):
            pltpu.make_async_copy(
                table_hbm.at[pl.ds(0, half)], bufs.at[b], gsems.at[b]
            ).wait()
            pltpu.sync_copy(
                bufs.at[b],
                out_hbm.at[
                    row0 + g // 2, pl.ds((g % 2) * half, half), pl.ds(0, d_model)
                ],
            )

        def body(t, carry):
            for b in range(_NBUF):
                g = t * _NBUF + b
                drain_store(b, g)
                pltpu.async_copy(
                    table_hbm.at[idx_v.at[g + _NBUF]], bufs.at[b], gsems.at[b]
                )
            return carry

        lax.fori_loop(0, n_main, body, 0)

        for b in range(_NBUF):
            drain_store(b, n_main * _NBUF + b)

    return k(table, idx3)


def kernel(token_ids, weight):
    b0, s0 = token_ids.shape
    vocab, d_model = weight.shape
    rows_per_w = b0 // _NW
    wide = _widen_tc(weight.T)
    table2 = wide.reshape(vocab * _LANES // d_model, d_model)
    idx3 = (token_ids * 2).reshape(_NW, 2 * rows_per_w, s0 // 2).astype(jnp.int32)
    out2 = _gather_sc(
        table2, idx3, rows_per_w=rows_per_w, seq=s0, d_model=d_model
    )
    return out2[:, :, :d_model]


# speedup vs baseline: 2.7803x; 1.0510x over previous
import functools

import jax
import jax.numpy as jnp
from jax import lax
from jax.experimental import pallas as pl
from jax.experimental.pallas import tpu as pltpu
from jax.experimental.pallas import tpu_sc as plsc

_NC = 2
_NS = 16
_NW = _NC * _NS
_LANES = 128

_NBUF = 4


_WIDEN_BLK = 32768


def _widen_tc(wt):
    d_model, vocab = wt.shape
    hblk = _WIDEN_BLK // 2
    n_blocks = pl.cdiv(vocab, _WIDEN_BLK)

    def body(x_ref, o_ref):
        o_ref[:, pl.ds(0, d_model)] = x_ref[:, pl.ds(0, hblk)].T
        o_ref[:, pl.ds(d_model, d_model)] = x_ref[:, pl.ds(hblk, hblk)].T

    return pl.pallas_call(
        body,
        grid=(n_blocks,),
        in_specs=[pl.BlockSpec((d_model, _WIDEN_BLK), lambda j: (0, j))],
        out_specs=pl.BlockSpec((hblk, 2 * d_model), lambda j: (j, 0)),
        out_shape=jax.ShapeDtypeStruct((n_blocks * hblk, 2 * d_model), jnp.float32),
    )(wt)


def _gather_sc(table, idx3, *, rows_per_w, seq, d_model):
    half = seq // 2
    n_groups = 2 * rows_per_w
    n_rows = _NW * rows_per_w
    mesh = plsc.VectorSubcoreMesh(core_axis_name="c", subcore_axis_name="s")
    n_main = n_groups // _NBUF - 1

    @functools.partial(
        pl.kernel,
        out_type=jax.ShapeDtypeStruct((n_rows, seq, 2 * d_model), jnp.float32),
        mesh=mesh,
        scratch_types=[
            pltpu.VMEM((n_groups, half), jnp.int32),
            pltpu.VMEM((_NBUF, half, d_model), jnp.float32),
            pltpu.SemaphoreType.DMA((_NBUF,)),
        ],
        compiler_params=pltpu.CompilerParams(use_tc_tiling_on_sc=False),
    )
    def k(table_hbm, idx_hbm, out_hbm, idx_v, bufs, gsems):
        wid = lax.axis_index("s") * _NC + lax.axis_index("c")
        row0 = wid * rows_per_w
        pltpu.sync_copy(idx_hbm.at[wid], idx_v)

        for b in range(_NBUF):
            pltpu.async_copy(table_hbm.at[idx_v.at[b]], bufs.at[b], gsems.at[b])

        def drain_store(b, g):
            pltpu.make_async_copy(
                table_hbm.at[pl.ds(0, half)], bufs.at[b], gsems.at[b]
            ).wait()
            pltpu.sync_copy(
                bufs.at[b],
                out_hbm.at[
                    row0 + g // 2, pl.ds((g % 2) * half, half), pl.ds(0, d_model)
                ],
            )

        def body(t, carry):
            for b in range(_NBUF):
                g = t * _NBUF + b
                drain_store(b, g)
                pltpu.async_copy(
                    table_hbm.at[idx_v.at[g + _NBUF]], bufs.at[b], gsems.at[b]
                )
            return carry

        lax.fori_loop(0, n_main, body, 0)

        for b in range(_NBUF):
            drain_store(b, n_main * _NBUF + b)

    return k(table, idx3)


def kernel(token_ids, weight):
    b0, s0 = token_ids.shape
    vocab, d_model = weight.shape
    rows_per_w = b0 // _NW
    wide = _widen_tc(weight.T)
    table2 = wide.reshape(-1, d_model)
    ids = token_ids.astype(jnp.int32)
    u = ids & (_WIDEN_BLK - 1)
    ids = (ids - u) + ((u << 1) & (_WIDEN_BLK - 1)) + (u >> 14)
    idx3 = ids.reshape(_NW, 2 * rows_per_w, s0 // 2)
    out2 = _gather_sc(
        table2, idx3, rows_per_w=rows_per_w, seq=s0, d_model=d_model
    )
    return out2[:, :, :d_model]
